# baseline (speedup 1.0000x reference)
import functools

import jax
import jax.numpy as jnp
from jax import lax
from jax.experimental import pallas as pl
from jax.experimental.pallas import tpu as pltpu
from jax.experimental.pallas import tpu_sc as plsc

_VOCAB = 1000000
_DIM = 32
_BATCH = 4096
_HIST = 50
_B = _BATCH * _HIST

_NC = 2
_NS = 16
_NW = _NC * _NS
_BPW = _B // _NW
_CH = 1600
_STEPS = _BPW // _CH

_mesh = plsc.VectorSubcoreMesh(core_axis_name="c", subcore_axis_name="s")


@functools.partial(
    pl.kernel,
    out_type=(
        jax.ShapeDtypeStruct((_B, _DIM), jnp.float32),
        jax.ShapeDtypeStruct((_B, _DIM), jnp.float32),
    ),
    mesh=_mesh,
    scratch_types=[
        pltpu.VMEM((_CH,), jnp.int32),
        pltpu.VMEM((_CH, _DIM), jnp.float32),
        pltpu.VMEM((_CH, _DIM), jnp.float32),
        pltpu.SemaphoreType.DMA,
        pltpu.SemaphoreType.DMA,
    ],
    compiler_params=pltpu.CompilerParams(use_tc_tiling_on_sc=False),
)
def _gather_both(ids_hbm, real_hbm, imag_hbm, out_r_hbm, out_i_hbm,
                 idx_v, rows_r, rows_i, sem_r, sem_i):
    wid = lax.axis_index("s") * _NC + lax.axis_index("c")
    base = wid * _BPW

    def step(i, carry):
        off = base + i * _CH
        pltpu.sync_copy(ids_hbm.at[pl.ds(off, _CH)], idx_v)
        cp_r = pltpu.async_copy(real_hbm.at[idx_v], rows_r, sem_r)
        cp_i = pltpu.async_copy(imag_hbm.at[idx_v], rows_i, sem_i)
        cp_r.wait()
        cp_i.wait()
        pltpu.sync_copy(rows_r, out_r_hbm.at[pl.ds(off, _CH)])
        pltpu.sync_copy(rows_i, out_i_hbm.at[pl.ds(off, _CH)])
        return carry

    lax.fori_loop(0, _STEPS, step, 0)


def kernel(input_ids, emb_real, emb_imag):
    ids = input_ids.reshape(-1).astype(jnp.int32)
    out_r, out_i = _gather_both(ids, emb_real, emb_imag)
    real = out_r.reshape(_BATCH, _HIST, _DIM)
    imag = out_i.reshape(_BATCH, _HIST, _DIM)
    return lax.complex(real, imag)


# speedup vs baseline: 1.5230x; 1.5230x over previous
import functools

import jax
import jax.numpy as jnp
from jax import lax
from jax.experimental import pallas as pl
from jax.experimental.pallas import tpu as pltpu
from jax.experimental.pallas import tpu_sc as plsc

_VOCAB = 1000000
_DIM = 32
_BATCH = 4096
_HIST = 50
_B = _BATCH * _HIST

_NC = 2
_NS = 16
_NW = _NC * _NS
_NBT = _BATCH // 128
_UNITS = _HIST * _NBT
_UPW = _UNITS // _NW

_mesh = plsc.VectorSubcoreMesh(core_axis_name="c", subcore_axis_name="s")


@functools.partial(
    pl.kernel,
    out_type=(
        jax.ShapeDtypeStruct((_HIST, _DIM // 8, _NBT, 8, 128), jnp.float32),
        jax.ShapeDtypeStruct((_HIST, _DIM // 8, _NBT, 8, 128), jnp.float32),
    ),
    mesh=_mesh,
    scratch_types=[
        pltpu.VMEM((128,), jnp.int32),
        pltpu.VMEM((128, _DIM), jnp.float32),
        pltpu.VMEM((128, _DIM), jnp.float32),
        pltpu.VMEM((_DIM, 128), jnp.float32),
        pltpu.VMEM((_DIM, 128), jnp.float32),
        pltpu.SemaphoreType.DMA,
        pltpu.SemaphoreType.DMA,
        pltpu.SemaphoreType.DMA,
    ],
    compiler_params=pltpu.CompilerParams(
        use_tc_tiling_on_sc=False, needs_layout_passes=False),
)
def _gather_both(ids_hbm, real_hbm, imag_hbm, out_r_hbm, out_i_hbm,
                 idx_v, rows_r, rows_i, out_tr, out_ti, sem_r, sem_i, sem_t):
    wid = lax.axis_index("s") * _NC + lax.axis_index("c")
    ubase = wid * _UPW

    def unit(k, carry):
        u = ubase + k
        h = u // _NBT
        bt = u % _NBT
        pltpu.sync_copy(ids_hbm.at[h, pl.ds(bt * 128, 128)], idx_v)
        cp_r = pltpu.async_copy(real_hbm.at[idx_v], rows_r, sem_r)
        cp_i = pltpu.async_copy(imag_hbm.at[idx_v], rows_i, sem_i)
        cp_r.wait()
        cp_i.wait()
        iota16 = lax.iota(jnp.int32, 16)

        def trow(c, tc):
            cvec = jnp.full((16,), c, dtype=jnp.int32)
            for dhalf in range(_DIM // 16):
                dlanes = iota16 + (16 * dhalf)
                vr = rows_r[c, pl.ds(16 * dhalf, 16)]
                vi = rows_i[c, pl.ds(16 * dhalf, 16)]
                plsc.store_scatter(out_tr, [dlanes, cvec], vr)
                plsc.store_scatter(out_ti, [dlanes, cvec], vi)
            return tc

        lax.fori_loop(0, 128, trow, 0)
        ocps = []
        for dt in range(_DIM // 8):
            ocps.append(pltpu.async_copy(
                out_tr.at[pl.ds(dt * 8, 8)], out_r_hbm.at[h, dt, bt], sem_r))
            ocps.append(pltpu.async_copy(
                out_ti.at[pl.ds(dt * 8, 8)], out_i_hbm.at[h, dt, bt], sem_i))
        for cp in ocps:
            cp.wait()
        return carry

    lax.fori_loop(0, _UPW, unit, 0)


def kernel(input_ids, emb_real, emb_imag):
    ids_t = input_ids.T.astype(jnp.int32)
    o_r, o_i = _gather_both(ids_t, emb_real, emb_imag)
    p_r = o_r.transpose(0, 1, 3, 2, 4).reshape(_HIST, _DIM, _BATCH)
    p_i = o_i.transpose(0, 1, 3, 2, 4).reshape(_HIST, _DIM, _BATCH)
    out_t = lax.complex(p_r, p_i)
    return out_t.transpose(2, 0, 1)
